# baseline (speedup 1.0000x reference)
import functools

import jax
import jax.numpy as jnp
from jax import lax
from jax.experimental import pallas as pl
from jax.experimental.pallas import tpu as pltpu
from jax.experimental.pallas import tpu_sc as plsc

AVG_NEIGH = 32.0


def _up_body(x_ref, w_ref, o_ref):
    o_ref[...] = jnp.dot(x_ref[...], w_ref[...],
                         preferred_element_type=jnp.float32)


def _mod_body(re_ref, wr1_ref, wr2_ref, wedge_ref, o_ref):
    d_rad = wr1_ref.shape[0]
    d_edge = wedge_ref.shape[0]
    blk = re_ref[...]
    t = jnp.dot(blk[:, :d_rad], wr1_ref[...],
                preferred_element_type=jnp.float32)
    t = t * jax.nn.sigmoid(t)
    rw = jnp.dot(t, wr2_ref[...], preferred_element_type=jnp.float32)
    ew = jnp.dot(blk[:, d_rad:d_rad + d_edge], wedge_ref[...],
                 preferred_element_type=jnp.float32)
    o_ref[...] = rw * ew


def _down_body(a_ref, b_ref, w_ref, o_ref):
    a = (a_ref[0] + a_ref[1] + b_ref[0] + b_ref[1]) * (1.0 / AVG_NEIGH)
    o_ref[...] = jnp.dot(a, w_ref[...], preferred_element_type=jnp.float32)


def _linear_up(node_features, w_up):
    n, d = node_features.shape
    bn = 1000
    return pl.pallas_call(
        _up_body,
        grid=(n // bn,),
        in_specs=[
            pl.BlockSpec((bn, d), lambda i: (i, 0)),
            pl.BlockSpec((d, d), lambda i: (0, 0)),
        ],
        out_specs=pl.BlockSpec((bn, d), lambda i: (i, 0)),
        out_shape=jax.ShapeDtypeStruct((n, d), jnp.float32),
    )(node_features, w_up)


def _edge_mod(re, w_r1, w_r2, w_edge, e_lo, e_count):
    d_re = re.shape[1]
    d_rad = w_r1.shape[0]
    hid = w_r1.shape[1]
    d = w_r2.shape[1]
    d_edge = w_edge.shape[0]
    be = 4000
    lo_blk = e_lo // be
    assert e_lo % be == 0 and e_count % be == 0
    return pl.pallas_call(
        _mod_body,
        grid=(e_count // be,),
        in_specs=[
            pl.BlockSpec((be, d_re), lambda i: (i + lo_blk, 0)),
            pl.BlockSpec((d_rad, hid), lambda i: (0, 0)),
            pl.BlockSpec((hid, d), lambda i: (0, 0)),
            pl.BlockSpec((d_edge, d), lambda i: (0, 0)),
        ],
        out_specs=pl.BlockSpec((be, d), lambda i: (i, 0)),
        out_shape=jax.ShapeDtypeStruct((e_count, d), jnp.float32),
    )(re, w_r1, w_r2, w_edge)


def _linear_down2(agg_a, agg_b, w_down):
    _, n, d = agg_a.shape
    bn = 1000
    return pl.pallas_call(
        _down_body,
        grid=(n // bn,),
        in_specs=[
            pl.BlockSpec((2, bn, d), lambda i: (0, i, 0)),
            pl.BlockSpec((2, bn, d), lambda i: (0, i, 0)),
            pl.BlockSpec((d, d), lambda i: (0, 0)),
        ],
        out_specs=pl.BlockSpec((bn, d), lambda i: (i, 0)),
        out_shape=jax.ShapeDtypeStruct((n, d), jnp.float32),
    )(agg_a, agg_b, w_down)


_K = 80
_ZROWS = 80
_WROWS = 200


def _sc_scatter(h, mod, sr, e_lo, e_count):
    n, d = h.shape
    info = plsc.get_sparse_core_info()
    nc, ns = info.num_cores, info.num_subcores
    nw = nc * ns
    e_per_tile = e_count // nw
    assert e_per_tile * nw == e_count and e_per_tile % _K == 0
    n_chunks = e_per_tile // _K
    n_zero_blocks = n // _ZROWS
    n_wb_blocks = n // _WROWS
    assert n_zero_blocks * _ZROWS == n and n_wb_blocks * _WROWS == n

    odd = n_chunks % 2 == 1
    assert n_chunks >= 4
    n_pairs_main = (n_chunks - 3) // 2 if odd else (n_chunks - 2) // 2

    mesh = plsc.VectorSubcoreMesh(core_axis_name="c", subcore_axis_name="s",
                                  num_cores=nc, num_subcores=ns)

    @functools.partial(
        pl.kernel,
        mesh=mesh,
        out_type=jax.ShapeDtypeStruct((nc, n, d), jnp.float32),
        scratch_types=[
            pltpu.VMEM((2 * _K,), jnp.int32),
            pltpu.VMEM((2 * _K,), jnp.int32),
            pltpu.VMEM((_K,), jnp.int32),
            pltpu.VMEM((_K,), jnp.int32),
            pltpu.VMEM((_K, d), jnp.float32),
            pltpu.VMEM((_K, d), jnp.float32),
            pltpu.VMEM((_K, d), jnp.float32),
            pltpu.VMEM((_K, d), jnp.float32),
            pltpu.VMEM_SHARED((n, d), jnp.float32),
            pltpu.SemaphoreType.DMA,
            pltpu.SemaphoreType.DMA,
            pltpu.SemaphoreType.DMA,
            pltpu.SemaphoreType.DMA,
            pltpu.SemaphoreType.DMA,
            pltpu.SemaphoreType.DMA,
        ],
    )
    def body(h_hbm, mod_hbm, sr_hbm, out_hbm,
             srb0, srb1, ridx0, ridx1, hrows0, hrows1, mrows0, mrows1,
             agg,
             in0, in1, g0, g1, s0, s1):
        cid = lax.axis_index("c")
        sid = lax.axis_index("s")
        wid = cid * ns + sid

        srb = (srb0, srb1)
        ridx = (ridx0, ridx1)
        hrows = (hrows0, hrows1)
        mrows = (mrows0, mrows1)
        insem = (in0, in1)
        gsem = (g0, g1)
        ssem = (s0, s1)

        zero = jnp.zeros((16,), jnp.float32)

        def zfill(i, carry):
            for c in range(d // 16):
                mrows0[i, pl.ds(c * 16, 16)] = zero
            return carry

        lax.fori_loop(0, _ZROWS, zfill, 0)
        for b in range(n_zero_blocks):
            @pl.when(b % ns == sid)
            def _():
                pltpu.sync_copy(mrows0, agg.at[pl.ds(b * _ZROWS, _ZROWS)])
        plsc.subcore_barrier()

        mbase0 = wid * e_per_tile
        ebase0 = e_lo + wid * e_per_tile

        def issue_loads(off, b):
            pltpu.async_copy(sr_hbm.at[pl.ds(2 * (ebase0 + off), 2 * _K)],
                             srb[b], insem[b])
            pltpu.async_copy(mod_hbm.at[pl.ds(mbase0 + off, _K)], mrows[b],
                             insem[b])

        def wait_loads(off, b):
            pltpu.make_async_copy(sr_hbm.at[pl.ds(2 * (ebase0 + off), 2 * _K)],
                                  srb[b], insem[b]).wait()
            pltpu.make_async_copy(mod_hbm.at[pl.ds(mbase0 + off, _K)],
                                  mrows[b], insem[b]).wait()

        def issue_gather(b):
            pltpu.async_copy(h_hbm.at[srb[b].at[pl.ds(0, _K)]], hrows[b],
                             gsem[b])

        def wait_gather(b):
            pltpu.make_async_copy(h_hbm.at[srb[b].at[pl.ds(0, _K)]], hrows[b],
                                  gsem[b]).wait()

        def mul(b):
            for t in range(_K // 16):
                ridx[b][pl.ds(t * 16, 16)] = srb[b][pl.ds(_K + t * 16, 16)]

            @plsc.parallel_loop(0, _K, 1, unroll=4)
            def mrow(i):
                for c in range(d // 16):
                    sl = pl.ds(c * 16, 16)
                    mrows[b][i, sl] = mrows[b][i, sl] * hrows[b][i, sl]

        def issue_scatter(b):
            pltpu.async_copy(mrows[b], agg.at[ridx[b]], ssem[b], add=True)

        def wait_scatter(b):
            pltpu.make_async_copy(mrows[b], agg.at[ridx[b]], ssem[b]).wait()

        issue_loads(0, 0)
        issue_loads(_K, 1)
        wait_loads(0, 0)
        issue_gather(0)

        def pair(i, carry):
            c0 = (2 * i) * _K
            wait_gather(0)
            mul(0)
            issue_scatter(0)
            wait_loads(c0 + _K, 1)
            issue_gather(1)
            wait_scatter(0)
            issue_loads(c0 + 2 * _K, 0)
            wait_gather(1)
            mul(1)
            issue_scatter(1)
            wait_loads(c0 + 2 * _K, 0)
            issue_gather(0)
            wait_scatter(1)
            issue_loads(c0 + 3 * _K, 1)
            return carry

        lax.fori_loop(0, n_pairs_main, pair, 0)

        if odd:
            cA = (n_chunks - 3) * _K
            wait_gather(0)
            mul(0)
            issue_scatter(0)
            wait_loads(cA + _K, 1)
            issue_gather(1)
            wait_scatter(0)
            issue_loads(cA + 2 * _K, 0)
            wait_gather(1)
            mul(1)
            issue_scatter(1)
            wait_loads(cA + 2 * _K, 0)
            issue_gather(0)
            wait_scatter(1)
            wait_gather(0)
            mul(0)
            issue_scatter(0)
            wait_scatter(0)
        else:
            cB = (n_chunks - 2) * _K
            wait_gather(0)
            mul(0)
            issue_scatter(0)
            wait_loads(cB + _K, 1)
            issue_gather(1)
            wait_scatter(0)
            wait_gather(1)
            mul(1)
            issue_scatter(1)
            wait_scatter(1)
        plsc.subcore_barrier()

        for b in range(n_wb_blocks):
            @pl.when(b % ns == sid)
            def _():
                pltpu.sync_copy(agg.at[pl.ds(b * _WROWS, _WROWS)],
                                out_hbm.at[cid, pl.ds(b * _WROWS, _WROWS)])

    return body(h, mod, sr)


def kernel(node_features, edge_features, radial_embedding, senders, receivers,
           W_up, W_r1, W_r2, W_edge, W_down):
    e = senders.shape[0]
    e_a = (e * 3 // 5 // 4000) * 4000
    e_b = e - e_a
    h = _linear_up(node_features, W_up)
    d_re = radial_embedding.shape[1] + edge_features.shape[1]
    re = jnp.concatenate(
        [radial_embedding, edge_features,
         jnp.zeros((e, 128 - d_re), jnp.float32)], axis=1)
    sr = jnp.stack([senders.reshape(-1, _K), receivers.reshape(-1, _K)],
                   axis=1).reshape(-1)
    mod_a = _edge_mod(re, W_r1, W_r2, W_edge, 0, e_a)
    agg_a = _sc_scatter(h, mod_a, sr, 0, e_a)
    mod_b = _edge_mod(re, W_r1, W_r2, W_edge, e_a, e_b)
    agg_b = _sc_scatter(h, mod_b, sr, e_a, e_b)
    return _linear_down2(agg_a, agg_b, W_down)


# speedup vs baseline: 1.1609x; 1.1609x over previous
import functools

import jax
import jax.numpy as jnp
from jax import lax
from jax.experimental import pallas as pl
from jax.experimental.pallas import tpu as pltpu
from jax.experimental.pallas import tpu_sc as plsc

AVG_NEIGH = 32.0


def _up_body(x_ref, w_ref, o_ref):
    o_ref[...] = jnp.dot(x_ref[...], w_ref[...],
                         preferred_element_type=jnp.float32)


def _mod_body(re_ref, wr1_ref, wr2_ref, wedge_ref, o_ref):
    d_rad = wr1_ref.shape[0]
    d_edge = wedge_ref.shape[0]
    blk = re_ref[...]
    t = jnp.dot(blk[:, :d_rad], wr1_ref[...],
                preferred_element_type=jnp.float32)
    t = t * jax.nn.sigmoid(t)
    rw = jnp.dot(t, wr2_ref[...], preferred_element_type=jnp.float32)
    ew = jnp.dot(blk[:, d_rad:d_rad + d_edge], wedge_ref[...],
                 preferred_element_type=jnp.float32)
    o_ref[...] = rw * ew


def _down_body(a_ref, b_ref, w_ref, o_ref):
    a = (a_ref[0] + a_ref[1] + b_ref[0] + b_ref[1]) * (1.0 / AVG_NEIGH)
    o_ref[...] = jnp.dot(a, w_ref[...], preferred_element_type=jnp.float32)


def _linear_up(node_features, w_up):
    n, d = node_features.shape
    bn = 1000
    return pl.pallas_call(
        _up_body,
        grid=(n // bn,),
        in_specs=[
            pl.BlockSpec((bn, d), lambda i: (i, 0)),
            pl.BlockSpec((d, d), lambda i: (0, 0)),
        ],
        out_specs=pl.BlockSpec((bn, d), lambda i: (i, 0)),
        out_shape=jax.ShapeDtypeStruct((n, d), jnp.float32),
    )(node_features, w_up)


def _edge_mod(re, w_r1, w_r2, w_edge, e_lo, e_count):
    d_re = re.shape[1]
    d_rad = w_r1.shape[0]
    hid = w_r1.shape[1]
    d = w_r2.shape[1]
    d_edge = w_edge.shape[0]
    be = 4000
    lo_blk = e_lo // be
    assert e_lo % be == 0 and e_count % be == 0
    return pl.pallas_call(
        _mod_body,
        grid=(e_count // be,),
        in_specs=[
            pl.BlockSpec((be, d_re), lambda i: (i + lo_blk, 0)),
            pl.BlockSpec((d_rad, hid), lambda i: (0, 0)),
            pl.BlockSpec((hid, d), lambda i: (0, 0)),
            pl.BlockSpec((d_edge, d), lambda i: (0, 0)),
        ],
        out_specs=pl.BlockSpec((be, d), lambda i: (i, 0)),
        out_shape=jax.ShapeDtypeStruct((e_count, d), jnp.float32),
    )(re, w_r1, w_r2, w_edge)


def _linear_down2(agg_a, agg_b, w_down):
    _, n, d = agg_a.shape
    bn = 1000
    return pl.pallas_call(
        _down_body,
        grid=(n // bn,),
        in_specs=[
            pl.BlockSpec((2, bn, d), lambda i: (0, i, 0)),
            pl.BlockSpec((2, bn, d), lambda i: (0, i, 0)),
            pl.BlockSpec((d, d), lambda i: (0, 0)),
        ],
        out_specs=pl.BlockSpec((bn, d), lambda i: (i, 0)),
        out_shape=jax.ShapeDtypeStruct((n, d), jnp.float32),
    )(agg_a, agg_b, w_down)


_K = 80
_ZROWS = 80
_WROWS = 200


def _sc_scatter(h, mod, senders, receivers, e_lo, e_count):
    n, d = h.shape
    info = plsc.get_sparse_core_info()
    nc, ns = info.num_cores, info.num_subcores
    nw = nc * ns
    e_per_tile = e_count // nw
    assert e_per_tile * nw == e_count and e_per_tile % _K == 0
    n_chunks = e_per_tile // _K
    n_zero_blocks = n // _ZROWS
    n_wb_blocks = n // _WROWS
    assert n_zero_blocks * _ZROWS == n and n_wb_blocks * _WROWS == n

    odd = n_chunks % 2 == 1
    assert n_chunks >= 4
    n_pairs_main = (n_chunks - 3) // 2 if odd else (n_chunks - 2) // 2

    mesh = plsc.VectorSubcoreMesh(core_axis_name="c", subcore_axis_name="s",
                                  num_cores=nc, num_subcores=ns)

    @functools.partial(
        pl.kernel,
        mesh=mesh,
        out_type=jax.ShapeDtypeStruct((nc, n, d), jnp.float32),
        scratch_types=[
            pltpu.VMEM((_K,), jnp.int32),
            pltpu.VMEM((_K,), jnp.int32),
            pltpu.VMEM((_K,), jnp.int32),
            pltpu.VMEM((_K,), jnp.int32),
            pltpu.VMEM((_K, d), jnp.float32),
            pltpu.VMEM((_K, d), jnp.float32),
            pltpu.VMEM((_K, d), jnp.float32),
            pltpu.VMEM((_K, d), jnp.float32),
            pltpu.VMEM_SHARED((n, d), jnp.float32),
            pltpu.SemaphoreType.DMA,
            pltpu.SemaphoreType.DMA,
            pltpu.SemaphoreType.DMA,
            pltpu.SemaphoreType.DMA,
            pltpu.SemaphoreType.DMA,
            pltpu.SemaphoreType.DMA,
        ],
    )
    def body(h_hbm, mod_hbm, send_hbm, recv_hbm, out_hbm,
             sidx0, sidx1, ridx0, ridx1, hrows0, hrows1, mrows0, mrows1,
             agg,
             in0, in1, g0, g1, s0, s1):
        cid = lax.axis_index("c")
        sid = lax.axis_index("s")
        wid = cid * ns + sid

        sidx = (sidx0, sidx1)
        ridx = (ridx0, ridx1)
        hrows = (hrows0, hrows1)
        mrows = (mrows0, mrows1)
        insem = (in0, in1)
        gsem = (g0, g1)
        ssem = (s0, s1)

        zero = jnp.zeros((16,), jnp.float32)

        def zfill(i, carry):
            for c in range(d // 16):
                mrows0[i, pl.ds(c * 16, 16)] = zero
            return carry

        lax.fori_loop(0, _ZROWS, zfill, 0)
        for b in range(n_zero_blocks):
            @pl.when(b % ns == sid)
            def _():
                pltpu.sync_copy(mrows0, agg.at[pl.ds(b * _ZROWS, _ZROWS)])
        plsc.subcore_barrier()

        mbase0 = wid * e_per_tile
        ebase0 = e_lo + wid * e_per_tile

        def issue_loads(off, b):
            pltpu.async_copy(send_hbm.at[pl.ds(ebase0 + off, _K)], sidx[b],
                             insem[b])
            pltpu.async_copy(recv_hbm.at[pl.ds(ebase0 + off, _K)], ridx[b],
                             insem[b])
            pltpu.async_copy(mod_hbm.at[pl.ds(mbase0 + off, _K)], mrows[b],
                             insem[b])

        def wait_loads(off, b):
            pltpu.make_async_copy(send_hbm.at[pl.ds(ebase0 + off, _K)],
                                  sidx[b], insem[b]).wait()
            pltpu.make_async_copy(recv_hbm.at[pl.ds(ebase0 + off, _K)],
                                  ridx[b], insem[b]).wait()
            pltpu.make_async_copy(mod_hbm.at[pl.ds(mbase0 + off, _K)],
                                  mrows[b], insem[b]).wait()

        def issue_gather(b):
            pltpu.async_copy(h_hbm.at[sidx[b]], hrows[b], gsem[b])

        def wait_gather(b):
            pltpu.make_async_copy(h_hbm.at[sidx[b]], hrows[b], gsem[b]).wait()

        def mul(b):
            @plsc.parallel_loop(0, _K, 1, unroll=4)
            def mrow(i):
                for c in range(d // 16):
                    sl = pl.ds(c * 16, 16)
                    mrows[b][i, sl] = mrows[b][i, sl] * hrows[b][i, sl]

        def issue_scatter(b):
            pltpu.async_copy(mrows[b], agg.at[ridx[b]], ssem[b], add=True)

        def wait_scatter(b):
            pltpu.make_async_copy(mrows[b], agg.at[ridx[b]], ssem[b]).wait()

        issue_loads(0, 0)
        issue_loads(_K, 1)
        wait_loads(0, 0)
        issue_gather(0)

        def pair(i, carry):
            c0 = (2 * i) * _K
            wait_gather(0)
            mul(0)
            issue_scatter(0)
            wait_loads(c0 + _K, 1)
            issue_gather(1)
            wait_scatter(0)
            issue_loads(c0 + 2 * _K, 0)
            wait_gather(1)
            mul(1)
            issue_scatter(1)
            wait_loads(c0 + 2 * _K, 0)
            issue_gather(0)
            wait_scatter(1)
            issue_loads(c0 + 3 * _K, 1)
            return carry

        lax.fori_loop(0, n_pairs_main, pair, 0)

        if odd:
            cA = (n_chunks - 3) * _K
            wait_gather(0)
            mul(0)
            issue_scatter(0)
            wait_loads(cA + _K, 1)
            issue_gather(1)
            wait_scatter(0)
            issue_loads(cA + 2 * _K, 0)
            wait_gather(1)
            mul(1)
            issue_scatter(1)
            wait_loads(cA + 2 * _K, 0)
            issue_gather(0)
            wait_scatter(1)
            wait_gather(0)
            mul(0)
            issue_scatter(0)
            wait_scatter(0)
        else:
            cB = (n_chunks - 2) * _K
            wait_gather(0)
            mul(0)
            issue_scatter(0)
            wait_loads(cB + _K, 1)
            issue_gather(1)
            wait_scatter(0)
            wait_gather(1)
            mul(1)
            issue_scatter(1)
            wait_scatter(1)
        plsc.subcore_barrier()

        for b in range(n_wb_blocks):
            @pl.when(b % ns == sid)
            def _():
                pltpu.sync_copy(agg.at[pl.ds(b * _WROWS, _WROWS)],
                                out_hbm.at[cid, pl.ds(b * _WROWS, _WROWS)])

    return body(h, mod, senders, receivers)


def kernel(node_features, edge_features, radial_embedding, senders, receivers,
           W_up, W_r1, W_r2, W_edge, W_down):
    e = senders.shape[0]
    e_a = (e * 3 // 5 // 4000) * 4000
    e_b = e - e_a
    h = _linear_up(node_features, W_up)
    re = jnp.concatenate([radial_embedding, edge_features], axis=1)
    mod_a = _edge_mod(re, W_r1, W_r2, W_edge, 0, e_a)
    agg_a = _sc_scatter(h, mod_a, senders, receivers, 0, e_a)
    mod_b = _edge_mod(re, W_r1, W_r2, W_edge, e_a, e_b)
    agg_b = _sc_scatter(h, mod_b, senders, receivers, e_a, e_b)
    return _linear_down2(agg_a, agg_b, W_down)
